# baseline (speedup 1.0000x reference)
import functools

import jax
import jax.numpy as jnp
from jax.experimental import pallas as pl
from jax.experimental.pallas import tpu as pltpu


def _kernel_d(ya_ref, yb_ref, w_ref, oa_ref, ob_ref, *, inv_total, h, m):
    w = w_ref[...].astype(jnp.float32)
    hm = h // m
    for y_ref, o_ref in ((ya_ref, oa_ref), (yb_ref, ob_ref)):
        y = y_ref[...].astype(jnp.float32)
        parts = [jnp.sum(y[:, q * hm:(q + 1) * hm], axis=(1, 2, 3))
                 for q in range(m)]
        t = jnp.concatenate(parts, axis=-1)
        z = jax.lax.dot_general(t, w,
                                dimension_numbers=(((1,), (1,)), ((), ())),
                                preferred_element_type=jnp.float32)
        o_ref[:, 0, :] = (z * inv_total).astype(o_ref.dtype)


def kernel(x, proj_weight):
    b, c, h, w, m = x.shape
    out_c = proj_weight.shape[0]
    hw = h * w
    inv_total = 1.0 / float(m * hw)

    xt = jnp.transpose(x, (0, 2, 3, 4, 1))
    co = jnp.arange(c)
    wcat = jnp.concatenate(
        [proj_weight[:, (co * m + q) % c] for q in range(m)], axis=1)

    tb = 4
    n = b // (2 * tb)
    bs = pl.BlockSpec
    za, zb = pl.pallas_call(
        functools.partial(_kernel_d, inv_total=inv_total, h=h, m=m),
        out_shape=(jax.ShapeDtypeStruct((b // 2, 1, out_c), x.dtype),
                   jax.ShapeDtypeStruct((b // 2, 1, out_c), x.dtype)),
        grid_spec=pltpu.PrefetchScalarGridSpec(
            num_scalar_prefetch=0,
            grid=(n,),
            in_specs=[bs((tb, h, w, m, c), lambda bi: (bi, 0, 0, 0, 0)),
                      bs((tb, h, w, m, c), lambda bi, _n=n: (bi + _n, 0, 0, 0, 0)),
                      bs((out_c, m * c), lambda bi: (0, 0))],
            out_specs=(bs((tb, 1, out_c), lambda bi: (bi, 0, 0)),
                       bs((tb, 1, out_c), lambda bi: (bi, 0, 0)))),
        compiler_params=pltpu.CompilerParams(
            dimension_semantics=("parallel",),
            vmem_limit_bytes=48 * 1024 * 1024),
    )(xt, xt, wcat)

    z = jnp.concatenate([za, zb], axis=0)
    return jnp.swapaxes(z, 1, 2)
